# baseline (speedup 1.0000x reference)
import dataclasses
import functools

import jax
import jax.numpy as jnp
from jax import lax
from jax.experimental import pallas as pl
from jax.experimental.pallas import tpu as pltpu
from jax.experimental.pallas import tpu_sc as plsc

DIM = 128
H = 4
DH = DIM // H
SCALE = DH ** (-0.5)

NC = 2
NS = 16
LANES = 16

CHUNK = 128
PACK = 32
PACK_SHIFT = 5

NLOC = 5120
ACC_V_ROWS = 5248
ACC_E_ROWS = 168
RING = 280
SCAN = 128
N_GATHER = 11264


def _qkv_body(x_ref, wq_ref, wkv_ref, bq_ref, bkv_ref, q_ref, kv_ref):
    xb = x_ref[...]
    dn = (((1,), (1,)), ((), ()))
    q = lax.dot_general(xb, wq_ref[...], dn, preferred_element_type=jnp.float32)
    kv = lax.dot_general(xb, wkv_ref[...], dn, preferred_element_type=jnp.float32)
    q_ref[...] = (q + bq_ref[...]) * SCALE
    kv_ref[...] = (kv + bkv_ref[...]).astype(jnp.bfloat16)


def _edge_body(q_hbm, kv_hbm, s_hbm, t_hbm, outv_hbm, oute_hbm,
               scan_s, scan_t, ring_s, ring_t,
               s_idx, t_idx, l_idx, qbuf, kvbuf, msgv, exbuf, acc_e_loc,
               acc_v, sem_q, sem_kv, sem_s,
               *, scan_blocks):
    cid = lax.axis_index("c")
    sid = lax.axis_index("s")
    lo = cid * NLOC

    zeros16 = jnp.zeros((LANES,), jnp.float32)
    lane_iota = lax.iota(jnp.int32, LANES)
    lane0 = lane_iota == 0

    @pl.loop(0, CHUNK)
    def _(erow):
        @pl.loop(0, DIM, step=LANES)
        def _(c0):
            msgv[erow, pl.ds(c0, LANES)] = zeros16

    @pl.loop(0, ACC_E_ROWS)
    def _(erow):
        @pl.loop(0, DIM, step=LANES)
        def _(c0):
            acc_e_loc[erow, pl.ds(c0, LANES)] = zeros16

    vrows_per_tile = ACC_V_ROWS // NS
    vcopies = [(r, min(CHUNK, vrows_per_tile - r))
               for r in range(0, vrows_per_tile, CHUNK)]
    for r, nr in vcopies:
        pltpu.sync_copy(msgv.at[pl.ds(0, nr)],
                        acc_v.at[pl.ds(sid * vrows_per_tile + r, nr)])

    plsc.subcore_barrier()

    dummy16 = jnp.full((LANES,), NLOC, jnp.int32)
    @pl.loop(0, CHUNK, step=LANES)
    def _(g0):
        l_idx[pl.ds(g0, LANES)] = dummy16
    pltpu.async_copy(msgv, acc_v.at[l_idx], sem_s, add=True)

    def flush():
        pltpu.make_async_copy(msgv, acc_v.at[l_idx], sem_s).wait()

        @pl.loop(0, CHUNK, step=LANES)
        def _(g0):
            sv = ring_s[pl.ds(g0, LANES)]
            s_idx[pl.ds(g0, LANES)] = sv
            l_idx[pl.ds(g0, LANES)] = sv - lo
            t_idx[pl.ds(g0, LANES)] = ring_t[pl.ds(g0, LANES)]

        cps = []
        for j in range(4):
            cps.append(pltpu.async_copy(
                kv_hbm.at[t_idx.at[pl.ds(j * 32, 32)]],
                kvbuf.at[pl.ds(j * 32, 32)], sem_kv))
        for j in range(2):
            cps.append(pltpu.async_copy(
                q_hbm.at[s_idx.at[pl.ds(j * 64, 64)]],
                qbuf.at[pl.ds(j * 64, 64)], sem_q))
        for cp in cps:
            cp.wait()

        @pl.loop(0, CHUNK, unroll=4)
        def _(e):
            for h in range(H):
                qa = qbuf[e, pl.ds(h * DH, LANES)]
                qb = qbuf[e, pl.ds(h * DH + LANES, LANES)]
                kp = plsc.bitcast(kvbuf[e, pl.ds(h * LANES, LANES)],
                                  jnp.bfloat16)
                ka, kb = plsc.unpack(kp, format=plsc.PackFormat.INTERLEAVED)
                sh = jnp.sum(qa * ka + qb * kb)
                plsc.store_scatter(
                    exbuf,
                    [jnp.full((LANES,), h, jnp.int32),
                     jnp.full((LANES,), e, jnp.int32)],
                    jnp.full((LANES,), sh, jnp.float32), mask=lane0)
                exv = jnp.exp(jnp.full((LANES,), sh, jnp.float32))
                vp = plsc.bitcast(
                    kvbuf[e, pl.ds(DIM // 2 + h * LANES, LANES)], jnp.bfloat16)
                va, vb = plsc.unpack(vp, format=plsc.PackFormat.INTERLEAVED)
                msgv[e, pl.ds(2 * h * LANES, LANES)] = va * exv
                msgv[e, pl.ds((2 * h + 1) * LANES, LANES)] = vb * exv

        @pl.loop(0, CHUNK, step=LANES)
        def _(g0):
            e_idx = lane_iota + g0
            lv = l_idx[pl.ds(g0, LANES)]
            erow = lax.shift_right_logical(lv, PACK_SHIFT)
            colbase = (lv & (PACK - 1)) * H
            for h in range(H):
                ch = plsc.load_gather(
                    exbuf, [jnp.full((LANES,), h, jnp.int32), e_idx])
                plsc.addupdate_scatter(acc_e_loc, [erow, colbase + h],
                                       jnp.exp(ch))

        pltpu.async_copy(msgv, acc_v.at[l_idx], sem_s, add=True)

        for j in range(CHUNK // LANES):
            ring_s[pl.ds(j * LANES, LANES)] = (
                ring_s[pl.ds(CHUNK + j * LANES, LANES)])
            ring_t[pl.ds(j * LANES, LANES)] = (
                ring_t[pl.ds(CHUNK + j * LANES, LANES)])

    edges_per_tile = scan_blocks * SCAN
    base_e = sid * edges_per_tile

    @pl.loop(0, scan_blocks, init_carry=jnp.int32(0))
    def final_off(bi, off):
        e0 = base_e + bi * SCAN
        cp_s = pltpu.async_copy(s_hbm.at[pl.ds(e0, SCAN)], scan_s, sem_q)
        cp_t = pltpu.async_copy(t_hbm.at[pl.ds(e0, SCAN)], scan_t, sem_kv)
        cp_s.wait()
        cp_t.wait()

        @pl.loop(0, SCAN, step=CHUNK, init_carry=off)
        def off_blk(c0, off_sc):
            @pl.loop(0, CHUNK, step=LANES, init_carry=off_sc)
            def off2(g0, off_c):
                sv = scan_s[pl.ds(c0 + g0, LANES)]
                tv = scan_t[pl.ds(c0 + g0, LANES)]
                lv = sv - lo
                keep = (lv >= 0) & (lv < NLOC)
                plsc.store_compressed(ring_s.at[pl.ds(off_c, LANES)], sv,
                                      mask=keep)
                plsc.store_compressed(ring_t.at[pl.ds(off_c, LANES)], tv,
                                      mask=keep)
                cnt = jnp.max(plsc.all_reduce_population_count(keep))
                return off_c + cnt

            @pl.when(off2 >= CHUNK)
            def _():
                flush()

            return jnp.where(off2 >= CHUNK, off2 - CHUNK, off2)

        return off_blk

    dummy_s = jnp.full((LANES,), lo + NLOC, jnp.int32)
    zeros_i = jnp.zeros((LANES,), jnp.int32)
    for j in range(CHUNK // LANES + 1):
        ring_s[pl.ds(final_off + j * LANES, LANES)] = dummy_s
        ring_t[pl.ds(final_off + j * LANES, LANES)] = zeros_i
    flush()

    pltpu.make_async_copy(msgv, acc_v.at[l_idx], sem_s).wait()
    pltpu.sync_copy(acc_e_loc, oute_hbm.at[cid, sid])

    plsc.subcore_barrier()

    for r, nr in vcopies:
        r0 = sid * vrows_per_tile + r
        pltpu.sync_copy(acc_v.at[pl.ds(r0, nr)], msgv.at[pl.ds(0, nr)])
        pltpu.sync_copy(msgv.at[pl.ds(0, nr)], outv_hbm.at[cid, pl.ds(r0, nr)])


def _norm_body(acc_ref, ex_ref, o_ref):
    a = acc_ref[...]
    exs = jnp.sum(ex_ref[...], axis=0)
    for h in range(H):
        den = exs[:, h][:, None] + 1e-16
        o_ref[:, h * DH:(h + 1) * DH] = a[:, h * DH:(h + 1) * DH] / den


def kernel(x, edge_index, num_super, W_qkv, b_qkv):
    n = x.shape[0]
    e = edge_index.shape[1]
    n_pad = NC * NLOC
    scan_blocks = (e + NS * SCAN - 1) // (NS * SCAN)
    e_pad = scan_blocks * NS * SCAN

    x_pad = jnp.pad(x, ((0, N_GATHER - n), (0, 0)))
    wq, wkv = W_qkv[:DIM], W_qkv[DIM:]
    bq, bkv = b_qkv[:DIM].reshape(1, DIM), b_qkv[DIM:].reshape(1, 2 * DIM)

    blk = 1024
    q_arr, kv_arr = pl.pallas_call(
        _qkv_body,
        grid=(N_GATHER // blk,),
        in_specs=[
            pl.BlockSpec((blk, DIM), lambda i: (i, 0)),
            pl.BlockSpec((DIM, DIM), lambda i: (0, 0)),
            pl.BlockSpec((2 * DIM, DIM), lambda i: (0, 0)),
            pl.BlockSpec((1, DIM), lambda i: (0, 0)),
            pl.BlockSpec((1, 2 * DIM), lambda i: (0, 0)),
        ],
        out_specs=[
            pl.BlockSpec((blk, DIM), lambda i: (i, 0)),
            pl.BlockSpec((blk, 2 * DIM), lambda i: (i, 0)),
        ],
        out_shape=[
            jax.ShapeDtypeStruct((N_GATHER, DIM), jnp.float32),
            jax.ShapeDtypeStruct((N_GATHER, 2 * DIM), jnp.bfloat16),
        ],
    )(x_pad, wq, wkv, bq, bkv)

    s = jnp.concatenate(
        [edge_index[0], jnp.full((e_pad - e,), n_pad - 1, jnp.int32)])
    t = jnp.concatenate([edge_index[1], jnp.zeros((e_pad - e,), jnp.int32)])

    cp = pltpu.CompilerParams()
    if "needs_layout_passes" in pltpu.CompilerParams.__dataclass_fields__:
        cp = dataclasses.replace(cp, needs_layout_passes=False)
    mesh = plsc.VectorSubcoreMesh(
        core_axis_name="c", subcore_axis_name="s",
        num_cores=NC, num_subcores=NS)
    edge_kernel = pl.kernel(
        functools.partial(_edge_body, scan_blocks=scan_blocks),
        out_type=[
            jax.ShapeDtypeStruct((NC, ACC_V_ROWS, DIM), jnp.float32),
            jax.ShapeDtypeStruct((NC, NS, ACC_E_ROWS, DIM), jnp.float32),
        ],
        mesh=mesh,
        scratch_types=[
            pltpu.VMEM((SCAN,), jnp.int32),
            pltpu.VMEM((SCAN,), jnp.int32),
            pltpu.VMEM((RING,), jnp.int32),
            pltpu.VMEM((RING,), jnp.int32),
            pltpu.VMEM((CHUNK,), jnp.int32),
            pltpu.VMEM((CHUNK,), jnp.int32),
            pltpu.VMEM((CHUNK,), jnp.int32),
            pltpu.VMEM((CHUNK, DIM), jnp.float32),
            pltpu.VMEM((CHUNK, DIM), jnp.int32),
            pltpu.VMEM((CHUNK, DIM), jnp.float32),
            pltpu.VMEM((H, CHUNK), jnp.float32),
            pltpu.VMEM((ACC_E_ROWS, DIM), jnp.float32),
            pltpu.VMEM_SHARED((ACC_V_ROWS, DIM), jnp.float32),
            pltpu.SemaphoreType.DMA,
            pltpu.SemaphoreType.DMA,
            pltpu.SemaphoreType.DMA,
        ],
        compiler_params=cp,
    )
    qperm = []
    for h in range(H):
        qperm += [h * DH + 2 * k for k in range(LANES)]
        qperm += [h * DH + 2 * k + 1 for k in range(LANES)]
    q_perm = q_arr[:, jnp.array(qperm, dtype=jnp.int32)]
    kv_i32 = lax.bitcast_convert_type(
        kv_arr.reshape(N_GATHER, DIM, 2), jnp.int32)
    acc_v, acc_e = edge_kernel(q_perm, kv_i32, s, t)

    outv = jnp.concatenate([acc_v[0, :NLOC], acc_v[1, :NLOC]])
    ex_r = jnp.concatenate([
        acc_e[0, :, :NLOC // PACK].reshape(NS, NLOC, H),
        acc_e[1, :, :NLOC // PACK].reshape(NS, NLOC, H),
    ], axis=1)

    fblk = 512
    out_pad = pl.pallas_call(
        _norm_body,
        grid=(n_pad // fblk,),
        in_specs=[
            pl.BlockSpec((fblk, DIM), lambda i: (i, 0)),
            pl.BlockSpec((NS, fblk, H), lambda i: (0, i, 0)),
        ],
        out_specs=pl.BlockSpec((fblk, DIM), lambda i: (i, 0)),
        out_shape=jax.ShapeDtypeStruct((n_pad, DIM), jnp.float32),
    )(outv, ex_r)
    perm = []
    for h in range(H):
        block = list(range(h * DH, h * DH + DH))
        for k in range(LANES):
            perm.append(h * DH + k)
    inv = [0] * DIM
    for h in range(H):
        for k in range(LANES):
            inv[h * DH + 2 * k] = h * DH + k
            inv[h * DH + 2 * k + 1] = h * DH + LANES + k
    out_fixed = out_pad[:, jnp.array(inv, dtype=jnp.int32)]
    return out_fixed[:n]


# speedup vs baseline: 1.4421x; 1.4421x over previous
import dataclasses
import functools

import jax
import jax.numpy as jnp
from jax import lax
from jax.experimental import pallas as pl
from jax.experimental.pallas import tpu as pltpu
from jax.experimental.pallas import tpu_sc as plsc

DIM = 128
H = 4
DH = DIM // H
SCALE = DH ** (-0.5)

NC = 2
NS = 16
LANES = 16

CHUNK = 128
PACK = 32
PACK_SHIFT = 5

NLOC = 5120
ACC_V_ROWS = 5248
ACC_E_ROWS = 168
RING = 280
SCAN = 128
N_GATHER = 11264


def _qkv_body(x_ref, wq_ref, wkv_ref, bq_ref, bkv_ref, q_ref, kv_ref):
    xb = x_ref[...]
    dn = (((1,), (1,)), ((), ()))
    q = lax.dot_general(xb, wq_ref[...], dn, preferred_element_type=jnp.float32)
    kv = lax.dot_general(xb, wkv_ref[...], dn, preferred_element_type=jnp.float32)
    q_ref[...] = (q + bq_ref[...]) * SCALE
    kv_ref[...] = kv + bkv_ref[...]


def _edge_body(q_hbm, kv_hbm, s_hbm, t_hbm, outv_hbm, oute_hbm,
               scan_s, scan_t, ring_s, ring_t,
               s_idx, t_idx, l_idx, qbuf, kvbuf, msgv, exbuf, acc_e_loc,
               acc_v, sem_q, sem_kv, sem_q2, sem_kv2, sem_s,
               *, scan_blocks):
    cid = lax.axis_index("c")
    sid = lax.axis_index("s")
    lo = cid * NLOC

    zeros16 = jnp.zeros((LANES,), jnp.float32)
    lane_iota = lax.iota(jnp.int32, LANES)
    lane0 = lane_iota == 0

    @pl.loop(0, CHUNK)
    def _(erow):
        @pl.loop(0, DIM, step=LANES)
        def _(c0):
            msgv[erow, pl.ds(c0, LANES)] = zeros16

    @pl.loop(0, ACC_E_ROWS)
    def _(erow):
        @pl.loop(0, DIM, step=LANES)
        def _(c0):
            acc_e_loc[erow, pl.ds(c0, LANES)] = zeros16

    vrows_per_tile = ACC_V_ROWS // NS
    vcopies = [(r, min(CHUNK, vrows_per_tile - r))
               for r in range(0, vrows_per_tile, CHUNK)]
    for r, nr in vcopies:
        pltpu.sync_copy(msgv.at[pl.ds(0, nr)],
                        acc_v.at[pl.ds(sid * vrows_per_tile + r, nr)])

    plsc.subcore_barrier()

    dummy16 = jnp.full((LANES,), NLOC, jnp.int32)
    @pl.loop(0, CHUNK, step=LANES)
    def _(g0):
        l_idx[pl.ds(g0, LANES)] = dummy16
    pltpu.async_copy(msgv, acc_v.at[l_idx], sem_s, add=True)

    def flush():
        pltpu.make_async_copy(msgv, acc_v.at[l_idx], sem_s).wait()

        @pl.loop(0, CHUNK, step=LANES)
        def _(g0):
            sv = ring_s[pl.ds(g0, LANES)]
            s_idx[pl.ds(g0, LANES)] = sv
            l_idx[pl.ds(g0, LANES)] = sv - lo
            t_idx[pl.ds(g0, LANES)] = ring_t[pl.ds(g0, LANES)]

        cps_a, cps_b = [], []
        for j in range(2):
            cps_a.append(pltpu.async_copy(
                kv_hbm.at[t_idx.at[pl.ds(j * 32, 32)]],
                kvbuf.at[pl.ds(j * 32, 32)], sem_kv))
        cps_a.append(pltpu.async_copy(
            q_hbm.at[s_idx.at[pl.ds(0, 64)]], qbuf.at[pl.ds(0, 64)], sem_q))
        for j in range(2, 4):
            cps_b.append(pltpu.async_copy(
                kv_hbm.at[t_idx.at[pl.ds(j * 32, 32)]],
                kvbuf.at[pl.ds(j * 32, 32)], sem_kv2))
        cps_b.append(pltpu.async_copy(
            q_hbm.at[s_idx.at[pl.ds(64, 64)]], qbuf.at[pl.ds(64, 64)], sem_q2))
        for cp in cps_a:
            cp.wait()

        @pl.loop(0, CHUNK // 2, unroll=2)
        def _(e):
            prods = []
            for j in range(DIM // LANES):
                qj = qbuf[e, pl.ds(j * LANES, LANES)]
                kj = kvbuf[e, pl.ds(j * LANES, LANES)]
                prods.append(qj * kj)
            for h in range(H):
                sh = jnp.sum(prods[2 * h] + prods[2 * h + 1])
                plsc.store_scatter(
                    exbuf,
                    [jnp.full((LANES,), h, jnp.int32),
                     jnp.full((LANES,), e, jnp.int32)],
                    jnp.full((LANES,), sh, jnp.float32), mask=lane0)
                exv = jnp.exp(jnp.full((LANES,), sh, jnp.float32))
                v0 = kvbuf[e, pl.ds(DIM + 2 * h * LANES, LANES)]
                v1 = kvbuf[e, pl.ds(DIM + (2 * h + 1) * LANES, LANES)]
                msgv[e, pl.ds(2 * h * LANES, LANES)] = v0 * exv
                msgv[e, pl.ds((2 * h + 1) * LANES, LANES)] = v1 * exv

        for cp in cps_b:
            cp.wait()

        @pl.loop(CHUNK // 2, CHUNK, unroll=2)
        def _(e):
            prods = []
            for j in range(DIM // LANES):
                qj = qbuf[e, pl.ds(j * LANES, LANES)]
                kj = kvbuf[e, pl.ds(j * LANES, LANES)]
                prods.append(qj * kj)
            for h in range(H):
                sh = jnp.sum(prods[2 * h] + prods[2 * h + 1])
                plsc.store_scatter(
                    exbuf,
                    [jnp.full((LANES,), h, jnp.int32),
                     jnp.full((LANES,), e, jnp.int32)],
                    jnp.full((LANES,), sh, jnp.float32), mask=lane0)
                exv = jnp.exp(jnp.full((LANES,), sh, jnp.float32))
                v0 = kvbuf[e, pl.ds(DIM + 2 * h * LANES, LANES)]
                v1 = kvbuf[e, pl.ds(DIM + (2 * h + 1) * LANES, LANES)]
                msgv[e, pl.ds(2 * h * LANES, LANES)] = v0 * exv
                msgv[e, pl.ds((2 * h + 1) * LANES, LANES)] = v1 * exv

        @pl.loop(0, CHUNK, step=LANES)
        def _(g0):
            e_idx = lane_iota + g0
            lv = l_idx[pl.ds(g0, LANES)]
            erow = lax.shift_right_logical(lv, PACK_SHIFT)
            colbase = (lv & (PACK - 1)) * H
            for h in range(H):
                ch = plsc.load_gather(
                    exbuf, [jnp.full((LANES,), h, jnp.int32), e_idx])
                plsc.addupdate_scatter(acc_e_loc, [erow, colbase + h],
                                       jnp.exp(ch))

        pltpu.async_copy(msgv, acc_v.at[l_idx], sem_s, add=True)

        for j in range(CHUNK // LANES):
            ring_s[pl.ds(j * LANES, LANES)] = (
                ring_s[pl.ds(CHUNK + j * LANES, LANES)])
            ring_t[pl.ds(j * LANES, LANES)] = (
                ring_t[pl.ds(CHUNK + j * LANES, LANES)])

    edges_per_tile = scan_blocks * SCAN
    base_e = sid * edges_per_tile

    @pl.loop(0, scan_blocks, init_carry=jnp.int32(0))
    def final_off(bi, off):
        e0 = base_e + bi * SCAN
        cp_s = pltpu.async_copy(s_hbm.at[pl.ds(e0, SCAN)], scan_s, sem_q)
        cp_t = pltpu.async_copy(t_hbm.at[pl.ds(e0, SCAN)], scan_t, sem_kv)
        cp_s.wait()
        cp_t.wait()

        @pl.loop(0, SCAN, step=CHUNK, init_carry=off)
        def off_blk(c0, off_sc):
            @pl.loop(0, CHUNK, step=LANES, init_carry=off_sc)
            def off2(g0, off_c):
                sv = scan_s[pl.ds(c0 + g0, LANES)]
                tv = scan_t[pl.ds(c0 + g0, LANES)]
                lv = sv - lo
                keep = (lv >= 0) & (lv < NLOC)
                plsc.store_compressed(ring_s.at[pl.ds(off_c, LANES)], sv,
                                      mask=keep)
                plsc.store_compressed(ring_t.at[pl.ds(off_c, LANES)], tv,
                                      mask=keep)
                cnt = jnp.max(plsc.all_reduce_population_count(keep))
                return off_c + cnt

            @pl.when(off2 >= CHUNK)
            def _():
                flush()

            return jnp.where(off2 >= CHUNK, off2 - CHUNK, off2)

        return off_blk

    dummy_s = jnp.full((LANES,), lo + NLOC, jnp.int32)
    zeros_i = jnp.zeros((LANES,), jnp.int32)
    for j in range(CHUNK // LANES + 1):
        ring_s[pl.ds(final_off + j * LANES, LANES)] = dummy_s
        ring_t[pl.ds(final_off + j * LANES, LANES)] = zeros_i
    flush()

    pltpu.make_async_copy(msgv, acc_v.at[l_idx], sem_s).wait()
    pltpu.sync_copy(acc_e_loc, oute_hbm.at[cid, sid])

    plsc.subcore_barrier()

    for r, nr in vcopies:
        r0 = sid * vrows_per_tile + r
        pltpu.sync_copy(acc_v.at[pl.ds(r0, nr)], msgv.at[pl.ds(0, nr)])
        pltpu.sync_copy(msgv.at[pl.ds(0, nr)], outv_hbm.at[cid, pl.ds(r0, nr)])


def _norm_body(acc_ref, ex_ref, o_ref):
    a = acc_ref[...]
    exs = jnp.sum(ex_ref[...], axis=0)
    for h in range(H):
        den = exs[:, h][:, None] + 1e-16
        o_ref[:, h * DH:(h + 1) * DH] = a[:, h * DH:(h + 1) * DH] / den


def kernel(x, edge_index, num_super, W_qkv, b_qkv):
    n = x.shape[0]
    e = edge_index.shape[1]
    n_pad = NC * NLOC
    scan_blocks = (e + NS * SCAN - 1) // (NS * SCAN)
    e_pad = scan_blocks * NS * SCAN

    x_pad = jnp.pad(x, ((0, N_GATHER - n), (0, 0)))
    wq, wkv = W_qkv[:DIM], W_qkv[DIM:]
    bq, bkv = b_qkv[:DIM].reshape(1, DIM), b_qkv[DIM:].reshape(1, 2 * DIM)

    blk = 1024
    q_arr, kv_arr = pl.pallas_call(
        _qkv_body,
        grid=(N_GATHER // blk,),
        in_specs=[
            pl.BlockSpec((blk, DIM), lambda i: (i, 0)),
            pl.BlockSpec((DIM, DIM), lambda i: (0, 0)),
            pl.BlockSpec((2 * DIM, DIM), lambda i: (0, 0)),
            pl.BlockSpec((1, DIM), lambda i: (0, 0)),
            pl.BlockSpec((1, 2 * DIM), lambda i: (0, 0)),
        ],
        out_specs=[
            pl.BlockSpec((blk, DIM), lambda i: (i, 0)),
            pl.BlockSpec((blk, 2 * DIM), lambda i: (i, 0)),
        ],
        out_shape=[
            jax.ShapeDtypeStruct((N_GATHER, DIM), jnp.float32),
            jax.ShapeDtypeStruct((N_GATHER, 2 * DIM), jnp.float32),
        ],
    )(x_pad, wq, wkv, bq, bkv)

    s = jnp.concatenate(
        [edge_index[0], jnp.full((e_pad - e,), n_pad - 1, jnp.int32)])
    t = jnp.concatenate([edge_index[1], jnp.zeros((e_pad - e,), jnp.int32)])

    cp = pltpu.CompilerParams()
    if "needs_layout_passes" in pltpu.CompilerParams.__dataclass_fields__:
        cp = dataclasses.replace(cp, needs_layout_passes=False)
    mesh = plsc.VectorSubcoreMesh(
        core_axis_name="c", subcore_axis_name="s",
        num_cores=NC, num_subcores=NS)
    edge_kernel = pl.kernel(
        functools.partial(_edge_body, scan_blocks=scan_blocks),
        out_type=[
            jax.ShapeDtypeStruct((NC, ACC_V_ROWS, DIM), jnp.float32),
            jax.ShapeDtypeStruct((NC, NS, ACC_E_ROWS, DIM), jnp.float32),
        ],
        mesh=mesh,
        scratch_types=[
            pltpu.VMEM((SCAN,), jnp.int32),
            pltpu.VMEM((SCAN,), jnp.int32),
            pltpu.VMEM((RING,), jnp.int32),
            pltpu.VMEM((RING,), jnp.int32),
            pltpu.VMEM((CHUNK,), jnp.int32),
            pltpu.VMEM((CHUNK,), jnp.int32),
            pltpu.VMEM((CHUNK,), jnp.int32),
            pltpu.VMEM((CHUNK, DIM), jnp.float32),
            pltpu.VMEM((CHUNK, 2 * DIM), jnp.float32),
            pltpu.VMEM((CHUNK, DIM), jnp.float32),
            pltpu.VMEM((H, CHUNK), jnp.float32),
            pltpu.VMEM((ACC_E_ROWS, DIM), jnp.float32),
            pltpu.VMEM_SHARED((ACC_V_ROWS, DIM), jnp.float32),
            pltpu.SemaphoreType.DMA,
            pltpu.SemaphoreType.DMA,
            pltpu.SemaphoreType.DMA,
            pltpu.SemaphoreType.DMA,
            pltpu.SemaphoreType.DMA,
        ],
        compiler_params=cp,
    )
    acc_v, acc_e = edge_kernel(q_arr, kv_arr, s, t)

    outv = jnp.concatenate([acc_v[0, :NLOC], acc_v[1, :NLOC]])
    ex_r = jnp.concatenate([
        acc_e[0, :, :NLOC // PACK].reshape(NS, NLOC, H),
        acc_e[1, :, :NLOC // PACK].reshape(NS, NLOC, H),
    ], axis=1)

    fblk = 512
    out_pad = pl.pallas_call(
        _norm_body,
        grid=(n_pad // fblk,),
        in_specs=[
            pl.BlockSpec((fblk, DIM), lambda i: (i, 0)),
            pl.BlockSpec((NS, fblk, H), lambda i: (0, i, 0)),
        ],
        out_specs=pl.BlockSpec((fblk, DIM), lambda i: (i, 0)),
        out_shape=jax.ShapeDtypeStruct((n_pad, DIM), jnp.float32),
    )(outv, ex_r)
    return out_pad[:n]


# speedup vs baseline: 1.4488x; 1.0046x over previous
import dataclasses
import functools

import jax
import jax.numpy as jnp
from jax import lax
from jax.experimental import pallas as pl
from jax.experimental.pallas import tpu as pltpu
from jax.experimental.pallas import tpu_sc as plsc

DIM = 128
H = 4
DH = DIM // H
SCALE = DH ** (-0.5)

NC = 2
NS = 16
LANES = 16

CHUNK = 128
PACK = 32
PACK_SHIFT = 5

NLOC = 5120
ACC_V_ROWS = 5248
ACC_E_ROWS = 168
RING = 280
SCAN = 128
N_GATHER = 11264


def _qkv_body(x_ref, wq_ref, wkv_ref, bq_ref, bkv_ref, q_ref, kv_ref):
    xb = x_ref[...]
    dn = (((1,), (1,)), ((), ()))
    q = lax.dot_general(xb, wq_ref[...], dn, preferred_element_type=jnp.float32)
    kv = lax.dot_general(xb, wkv_ref[...], dn, preferred_element_type=jnp.float32)
    q_ref[...] = (q + bq_ref[...]) * SCALE
    kv_ref[...] = kv + bkv_ref[...]


def _edge_body(q_hbm, kv_hbm, s_hbm, t_hbm, outv_hbm, oute_hbm,
               scan_s, scan_t, ring_s, ring_t,
               s_idx, t_idx, l_idx, qbuf, kvbuf, msgv, exbuf, acc_e_loc,
               acc_v, sem_q, sem_kv, sem_s,
               *, scan_blocks):
    cid = lax.axis_index("c")
    sid = lax.axis_index("s")
    lo = cid * NLOC

    zeros16 = jnp.zeros((LANES,), jnp.float32)
    lane_iota = lax.iota(jnp.int32, LANES)
    lane0 = lane_iota == 0

    @pl.loop(0, CHUNK)
    def _(erow):
        @pl.loop(0, DIM, step=LANES)
        def _(c0):
            msgv[erow, pl.ds(c0, LANES)] = zeros16

    @pl.loop(0, ACC_E_ROWS)
    def _(erow):
        @pl.loop(0, DIM, step=LANES)
        def _(c0):
            acc_e_loc[erow, pl.ds(c0, LANES)] = zeros16

    vrows_per_tile = ACC_V_ROWS // NS
    vcopies = [(r, min(CHUNK, vrows_per_tile - r))
               for r in range(0, vrows_per_tile, CHUNK)]
    for r, nr in vcopies:
        pltpu.sync_copy(msgv.at[pl.ds(0, nr)],
                        acc_v.at[pl.ds(sid * vrows_per_tile + r, nr)])

    plsc.subcore_barrier()

    dummy16 = jnp.full((LANES,), NLOC, jnp.int32)
    @pl.loop(0, CHUNK, step=LANES)
    def _(g0):
        l_idx[pl.ds(g0, LANES)] = dummy16
    pltpu.async_copy(msgv, acc_v.at[l_idx], sem_s, add=True)

    def flush():
        pltpu.make_async_copy(msgv, acc_v.at[l_idx], sem_s).wait()

        @pl.loop(0, CHUNK, step=LANES)
        def _(g0):
            sv = ring_s[pl.ds(g0, LANES)]
            s_idx[pl.ds(g0, LANES)] = sv
            l_idx[pl.ds(g0, LANES)] = sv - lo
            t_idx[pl.ds(g0, LANES)] = ring_t[pl.ds(g0, LANES)]

        cps = []
        for j in range(4):
            cps.append(pltpu.async_copy(
                kv_hbm.at[t_idx.at[pl.ds(j * 32, 32)]],
                kvbuf.at[pl.ds(j * 32, 32)], sem_kv))
        for j in range(2):
            cps.append(pltpu.async_copy(
                q_hbm.at[s_idx.at[pl.ds(j * 64, 64)]],
                qbuf.at[pl.ds(j * 64, 64)], sem_q))
        for cp in cps:
            cp.wait()

        @pl.loop(0, CHUNK, unroll=4)
        def _(e):
            prods = []
            for j in range(DIM // LANES):
                qj = qbuf[e, pl.ds(j * LANES, LANES)]
                kj = kvbuf[e, pl.ds(j * LANES, LANES)]
                prods.append(qj * kj)
            for h in range(H):
                sh = jnp.sum(prods[2 * h] + prods[2 * h + 1])
                plsc.store_scatter(
                    exbuf,
                    [jnp.full((LANES,), h, jnp.int32),
                     jnp.full((LANES,), e, jnp.int32)],
                    jnp.full((LANES,), sh, jnp.float32), mask=lane0)
                exv = jnp.exp(jnp.full((LANES,), sh, jnp.float32))
                v0 = kvbuf[e, pl.ds(DIM + 2 * h * LANES, LANES)]
                v1 = kvbuf[e, pl.ds(DIM + (2 * h + 1) * LANES, LANES)]
                msgv[e, pl.ds(2 * h * LANES, LANES)] = v0 * exv
                msgv[e, pl.ds((2 * h + 1) * LANES, LANES)] = v1 * exv

        @pl.loop(0, CHUNK, step=LANES)
        def _(g0):
            e_idx = lane_iota + g0
            lv = l_idx[pl.ds(g0, LANES)]
            erow = lax.shift_right_logical(lv, PACK_SHIFT)
            colbase = (lv & (PACK - 1)) * H
            for h in range(H):
                ch = plsc.load_gather(
                    exbuf, [jnp.full((LANES,), h, jnp.int32), e_idx])
                plsc.addupdate_scatter(acc_e_loc, [erow, colbase + h],
                                       jnp.exp(ch))

        pltpu.async_copy(msgv, acc_v.at[l_idx], sem_s, add=True)

        for j in range(CHUNK // LANES):
            ring_s[pl.ds(j * LANES, LANES)] = (
                ring_s[pl.ds(CHUNK + j * LANES, LANES)])
            ring_t[pl.ds(j * LANES, LANES)] = (
                ring_t[pl.ds(CHUNK + j * LANES, LANES)])

    edges_per_tile = scan_blocks * SCAN
    base_e = sid * edges_per_tile

    @pl.loop(0, scan_blocks, init_carry=jnp.int32(0))
    def final_off(bi, off):
        e0 = base_e + bi * SCAN
        cp_s = pltpu.async_copy(s_hbm.at[pl.ds(e0, SCAN)], scan_s, sem_q)
        cp_t = pltpu.async_copy(t_hbm.at[pl.ds(e0, SCAN)], scan_t, sem_kv)
        cp_s.wait()
        cp_t.wait()

        @pl.loop(0, SCAN, step=CHUNK, init_carry=off)
        def off_blk(c0, off_sc):
            @pl.loop(0, CHUNK, step=LANES, init_carry=off_sc)
            def off2(g0, off_c):
                sv = scan_s[pl.ds(c0 + g0, LANES)]
                tv = scan_t[pl.ds(c0 + g0, LANES)]
                lv = sv - lo
                keep = (lv >= 0) & (lv < NLOC)
                plsc.store_compressed(ring_s.at[pl.ds(off_c, LANES)], sv,
                                      mask=keep)
                plsc.store_compressed(ring_t.at[pl.ds(off_c, LANES)], tv,
                                      mask=keep)
                cnt = jnp.max(plsc.all_reduce_population_count(keep))
                return off_c + cnt

            @pl.when(off2 >= CHUNK)
            def _():
                flush()

            return jnp.where(off2 >= CHUNK, off2 - CHUNK, off2)

        return off_blk

    dummy_s = jnp.full((LANES,), lo + NLOC, jnp.int32)
    zeros_i = jnp.zeros((LANES,), jnp.int32)
    for j in range(CHUNK // LANES + 1):
        ring_s[pl.ds(final_off + j * LANES, LANES)] = dummy_s
        ring_t[pl.ds(final_off + j * LANES, LANES)] = zeros_i
    flush()

    pltpu.make_async_copy(msgv, acc_v.at[l_idx], sem_s).wait()
    pltpu.sync_copy(acc_e_loc, oute_hbm.at[cid, sid])

    plsc.subcore_barrier()

    for r, nr in vcopies:
        r0 = sid * vrows_per_tile + r
        pltpu.sync_copy(acc_v.at[pl.ds(r0, nr)], msgv.at[pl.ds(0, nr)])
        pltpu.sync_copy(msgv.at[pl.ds(0, nr)], outv_hbm.at[cid, pl.ds(r0, nr)])


def _norm_body(acc_ref, ex_ref, o_ref):
    a = acc_ref[...]
    exs = jnp.sum(ex_ref[...], axis=0)
    for h in range(H):
        den = exs[:, h][:, None] + 1e-16
        o_ref[:, h * DH:(h + 1) * DH] = a[:, h * DH:(h + 1) * DH] / den


def kernel(x, edge_index, num_super, W_qkv, b_qkv):
    n = x.shape[0]
    e = edge_index.shape[1]
    n_pad = NC * NLOC
    scan_blocks = (e + NS * SCAN - 1) // (NS * SCAN)
    e_pad = scan_blocks * NS * SCAN

    x_pad = jnp.pad(x, ((0, N_GATHER - n), (0, 0)))
    wq, wkv = W_qkv[:DIM], W_qkv[DIM:]
    bq, bkv = b_qkv[:DIM].reshape(1, DIM), b_qkv[DIM:].reshape(1, 2 * DIM)

    blk = 1024
    q_arr, kv_arr = pl.pallas_call(
        _qkv_body,
        grid=(N_GATHER // blk,),
        in_specs=[
            pl.BlockSpec((blk, DIM), lambda i: (i, 0)),
            pl.BlockSpec((DIM, DIM), lambda i: (0, 0)),
            pl.BlockSpec((2 * DIM, DIM), lambda i: (0, 0)),
            pl.BlockSpec((1, DIM), lambda i: (0, 0)),
            pl.BlockSpec((1, 2 * DIM), lambda i: (0, 0)),
        ],
        out_specs=[
            pl.BlockSpec((blk, DIM), lambda i: (i, 0)),
            pl.BlockSpec((blk, 2 * DIM), lambda i: (i, 0)),
        ],
        out_shape=[
            jax.ShapeDtypeStruct((N_GATHER, DIM), jnp.float32),
            jax.ShapeDtypeStruct((N_GATHER, 2 * DIM), jnp.float32),
        ],
    )(x_pad, wq, wkv, bq, bkv)

    s = jnp.concatenate(
        [edge_index[0], jnp.full((e_pad - e,), n_pad - 1, jnp.int32)])
    t = jnp.concatenate([edge_index[1], jnp.zeros((e_pad - e,), jnp.int32)])

    cp = pltpu.CompilerParams()
    if "needs_layout_passes" in pltpu.CompilerParams.__dataclass_fields__:
        cp = dataclasses.replace(cp, needs_layout_passes=False)
    mesh = plsc.VectorSubcoreMesh(
        core_axis_name="c", subcore_axis_name="s",
        num_cores=NC, num_subcores=NS)
    edge_kernel = pl.kernel(
        functools.partial(_edge_body, scan_blocks=scan_blocks),
        out_type=[
            jax.ShapeDtypeStruct((NC, ACC_V_ROWS, DIM), jnp.float32),
            jax.ShapeDtypeStruct((NC, NS, ACC_E_ROWS, DIM), jnp.float32),
        ],
        mesh=mesh,
        scratch_types=[
            pltpu.VMEM((SCAN,), jnp.int32),
            pltpu.VMEM((SCAN,), jnp.int32),
            pltpu.VMEM((RING,), jnp.int32),
            pltpu.VMEM((RING,), jnp.int32),
            pltpu.VMEM((CHUNK,), jnp.int32),
            pltpu.VMEM((CHUNK,), jnp.int32),
            pltpu.VMEM((CHUNK,), jnp.int32),
            pltpu.VMEM((CHUNK, DIM), jnp.float32),
            pltpu.VMEM((CHUNK, 2 * DIM), jnp.float32),
            pltpu.VMEM((CHUNK, DIM), jnp.float32),
            pltpu.VMEM((H, CHUNK), jnp.float32),
            pltpu.VMEM((ACC_E_ROWS, DIM), jnp.float32),
            pltpu.VMEM_SHARED((ACC_V_ROWS, DIM), jnp.float32),
            pltpu.SemaphoreType.DMA,
            pltpu.SemaphoreType.DMA,
            pltpu.SemaphoreType.DMA,
        ],
        compiler_params=cp,
    )
    acc_v, acc_e = edge_kernel(q_arr, kv_arr, s, t)

    outv = jnp.concatenate([acc_v[0, :NLOC], acc_v[1, :NLOC]])
    ex_r = jnp.concatenate([
        acc_e[0, :, :NLOC // PACK].reshape(NS, NLOC, H),
        acc_e[1, :, :NLOC // PACK].reshape(NS, NLOC, H),
    ], axis=1)

    fblk = 512
    out_pad = pl.pallas_call(
        _norm_body,
        grid=(n_pad // fblk,),
        in_specs=[
            pl.BlockSpec((fblk, DIM), lambda i: (i, 0)),
            pl.BlockSpec((NS, fblk, H), lambda i: (0, i, 0)),
        ],
        out_specs=pl.BlockSpec((fblk, DIM), lambda i: (i, 0)),
        out_shape=jax.ShapeDtypeStruct((n_pad, DIM), jnp.float32),
    )(outv, ex_r)
    return out_pad[:n]
